# Initial kernel scaffold; baseline (speedup 1.0000x reference)
import jax
import jax.numpy as jnp
from jax.experimental import pallas as pl


def kernel(x, alphas, betas):
    raise NotImplementedError("write your pallas kernel here")


# speedup vs baseline: 1.1743x; 1.1743x over previous
import functools

import jax
import jax.numpy as jnp
from jax import lax
from jax.experimental import pallas as pl
from jax.experimental.pallas import tpu as pltpu
from jax.experimental.pallas import tpu_sc as plsc

NBINS = 8
ROWS = 32768
COLS = 2048
BLK = 1024
NBLK = ROWS // BLK
N_TOTAL = ROWS * COLS

NW = 32
PER_W = N_TOTAL // NW
S = 16384
NCHUNK = PER_W // S
L = 16


def _stats_body(x_ref, acc_ref):
    j = pl.program_id(0)

    @pl.when(j == 0)
    def _init():
        acc_ref[...] = jnp.zeros_like(acc_ref)

    blk = x_ref[...]
    g = blk.reshape(BLK // 8, 8, COLS)
    acc_ref[0] += jnp.sum(g, axis=0)
    acc_ref[1] += jnp.sum(g * g, axis=0)


def _sc_map_body(x_hbm, offv_hbm, invv_hbm, ta_hbm, tb_hbm, out_hbm,
                 xv, yv, tav, tbv, offv, invv):
    c = lax.axis_index("c")
    s = lax.axis_index("s")
    wid = s * 2 + c
    pltpu.sync_copy(ta_hbm, tav)
    pltpu.sync_copy(tb_hbm, tbv)
    pltpu.sync_copy(offv_hbm, offv)
    pltpu.sync_copy(invv_hbm, invv)
    off = offv[...]
    inv = invv[...]
    base = wid * PER_W

    def chunk(ci, carry):
        pltpu.sync_copy(x_hbm.at[pl.ds(base + ci * S, S)], xv)

        def grp(i, carry2):
            xg = xv[pl.ds(i * L, L)]
            u = (xg - off) * inv
            idx = jnp.clip(u.astype(jnp.int32), 0, NBINS - 1)
            a = plsc.load_gather(tav, [idx])
            b = plsc.load_gather(tbv, [idx])
            yv[pl.ds(i * L, L)] = a * xg + b
            return carry2

        lax.fori_loop(0, S // L, grp, 0)
        pltpu.sync_copy(yv, out_hbm.at[pl.ds(base + ci * S, S)])
        return carry

    lax.fori_loop(0, NCHUNK, chunk, 0)


@jax.jit
def kernel(x, alphas, betas):
    init_alpha = jnp.concatenate([
        jnp.zeros((NBINS // 2,), dtype=jnp.float32),
        jnp.ones((NBINS - NBINS // 2,), dtype=jnp.float32),
    ])
    x2 = x.reshape(ROWS, COLS)

    acc = pl.pallas_call(
        _stats_body,
        grid=(NBLK,),
        in_specs=[pl.BlockSpec((BLK, COLS), lambda j: (j, 0))],
        out_specs=pl.BlockSpec((2, 8, COLS), lambda j: (0, 0, 0)),
        out_shape=jax.ShapeDtypeStruct((2, 8, COLS), jnp.float32),
    )(x2)

    n = jnp.float32(N_TOTAL)
    ssum = jnp.sum(acc[0])
    ssq = jnp.sum(acc[1])
    mean = ssum / n
    var = (ssq - ssum * ssum / n) / (n - 1.0)
    std = jnp.sqrt(var)
    step = 6.0 * std / NBINS
    off = mean - 3.0 * std
    inv = 1.0 / step
    offv = jnp.full((L,), off, jnp.float32)
    invv = jnp.full((L,), inv, jnp.float32)
    tav = jnp.pad(alphas + init_alpha, (0, L - NBINS))
    tbv = jnp.pad(betas, (0, L - NBINS))

    mesh = plsc.VectorSubcoreMesh(
        core_axis_name="c", subcore_axis_name="s", num_cores=2, num_subcores=16)
    out = pl.kernel(
        _sc_map_body,
        out_type=jax.ShapeDtypeStruct((N_TOTAL,), jnp.float32),
        mesh=mesh,
        compiler_params=pltpu.CompilerParams(needs_layout_passes=False),
        scratch_types=[
            pltpu.VMEM((S,), jnp.float32),
            pltpu.VMEM((S,), jnp.float32),
            pltpu.VMEM((L,), jnp.float32),
            pltpu.VMEM((L,), jnp.float32),
            pltpu.VMEM((L,), jnp.float32),
            pltpu.VMEM((L,), jnp.float32),
        ],
    )(x.reshape(N_TOTAL), offv, invv, tav, tbv)
    return out.reshape(x.shape)


# speedup vs baseline: 2.5617x; 2.1815x over previous
import functools

import jax
import jax.numpy as jnp
from jax import lax
from jax.experimental import pallas as pl
from jax.experimental.pallas import tpu as pltpu
from jax.experimental.pallas import tpu_sc as plsc

NBINS = 8
ROWS = 32768
COLS = 2048
BLK = 1024
NBLK = ROWS // BLK
N_TOTAL = ROWS * COLS

NW = 32
PER_W = N_TOTAL // NW
S = 16384
NCHUNK = PER_W // S
L = 16


def _stats_body(x_ref, acc_ref):
    j = pl.program_id(0)

    @pl.when(j == 0)
    def _init():
        acc_ref[...] = jnp.zeros_like(acc_ref)

    blk = x_ref[...]
    g = blk.reshape(BLK // 8, 8, COLS)
    acc_ref[0] += jnp.sum(g, axis=0)
    acc_ref[1] += jnp.sum(g * g, axis=0)


def _sc_map_body(x_hbm, offv_hbm, invv_hbm, ta_hbm, tb_hbm, out_hbm,
                 xv0, xv1, yv0, yv1, tav, tbv, offv, invv,
                 si0, si1, so0, so1):
    c = lax.axis_index("c")
    s = lax.axis_index("s")
    wid = s * 2 + c
    pltpu.sync_copy(ta_hbm, tav)
    pltpu.sync_copy(tb_hbm, tbv)
    pltpu.sync_copy(offv_hbm, offv)
    pltpu.sync_copy(invv_hbm, invv)
    off = offv[...]
    inv = invv[...]
    base = wid * PER_W

    def compute(xv, yv):
        @plsc.parallel_loop(0, S // L, 1, unroll=8)
        def _grp(i):
            xg = xv[pl.ds(i * L, L)]
            u = (xg - off) * inv
            idx = jnp.clip(u.astype(jnp.int32), 0, NBINS - 1)
            a = plsc.load_gather(tav, [idx])
            b = plsc.load_gather(tbv, [idx])
            yv[pl.ds(i * L, L)] = a * xg + b

    pltpu.async_copy(x_hbm.at[pl.ds(base, S)], xv0, si0)
    pltpu.async_copy(x_hbm.at[pl.ds(base + S, S)], xv1, si1)

    def stage(g, cidx, xv, yv, si, so):
        pltpu.make_async_copy(x_hbm.at[pl.ds(base, S)], xv, si).wait()

        @pl.when(g > 0)
        def _drain_out():
            pltpu.make_async_copy(yv, out_hbm.at[pl.ds(base, S)], so).wait()

        compute(xv, yv)
        pltpu.async_copy(yv, out_hbm.at[pl.ds(base + cidx * S, S)], so)

        @pl.when(cidx + 2 < NCHUNK)
        def _next_in():
            pltpu.async_copy(x_hbm.at[pl.ds(base + (cidx + 2) * S, S)], xv, si)

    def pair(g, carry):
        stage(g, 2 * g, xv0, yv0, si0, so0)
        stage(g, 2 * g + 1, xv1, yv1, si1, so1)
        return carry

    lax.fori_loop(0, NCHUNK // 2, pair, 0)
    pltpu.make_async_copy(yv0, out_hbm.at[pl.ds(base, S)], so0).wait()
    pltpu.make_async_copy(yv1, out_hbm.at[pl.ds(base, S)], so1).wait()


@jax.jit
def kernel(x, alphas, betas):
    init_alpha = jnp.concatenate([
        jnp.zeros((NBINS // 2,), dtype=jnp.float32),
        jnp.ones((NBINS - NBINS // 2,), dtype=jnp.float32),
    ])
    x2 = x.reshape(ROWS, COLS)

    acc = pl.pallas_call(
        _stats_body,
        grid=(NBLK,),
        in_specs=[pl.BlockSpec((BLK, COLS), lambda j: (j, 0))],
        out_specs=pl.BlockSpec((2, 8, COLS), lambda j: (0, 0, 0)),
        out_shape=jax.ShapeDtypeStruct((2, 8, COLS), jnp.float32),
    )(x2)

    n = jnp.float32(N_TOTAL)
    ssum = jnp.sum(acc[0])
    ssq = jnp.sum(acc[1])
    mean = ssum / n
    var = (ssq - ssum * ssum / n) / (n - 1.0)
    std = jnp.sqrt(var)
    step = 6.0 * std / NBINS
    off = mean - 3.0 * std
    inv = 1.0 / step
    offv = jnp.full((L,), off, jnp.float32)
    invv = jnp.full((L,), inv, jnp.float32)
    tav = jnp.pad(alphas + init_alpha, (0, L - NBINS))
    tbv = jnp.pad(betas, (0, L - NBINS))

    mesh = plsc.VectorSubcoreMesh(
        core_axis_name="c", subcore_axis_name="s", num_cores=2, num_subcores=16)
    out = pl.kernel(
        _sc_map_body,
        out_type=jax.ShapeDtypeStruct((N_TOTAL,), jnp.float32),
        mesh=mesh,
        compiler_params=pltpu.CompilerParams(needs_layout_passes=False),
        scratch_types=[
            pltpu.VMEM((S,), jnp.float32),
            pltpu.VMEM((S,), jnp.float32),
            pltpu.VMEM((S,), jnp.float32),
            pltpu.VMEM((S,), jnp.float32),
            pltpu.VMEM((L,), jnp.float32),
            pltpu.VMEM((L,), jnp.float32),
            pltpu.VMEM((L,), jnp.float32),
            pltpu.VMEM((L,), jnp.float32),
            pltpu.SemaphoreType.DMA,
            pltpu.SemaphoreType.DMA,
            pltpu.SemaphoreType.DMA,
            pltpu.SemaphoreType.DMA,
        ],
    )(x.reshape(N_TOTAL), offv, invv, tav, tbv)
    return out.reshape(x.shape)


# speedup vs baseline: 2.9017x; 1.1327x over previous
import functools

import jax
import jax.numpy as jnp
from jax import lax
from jax.experimental import pallas as pl
from jax.experimental.pallas import tpu as pltpu
from jax.experimental.pallas import tpu_sc as plsc

NBINS = 8
ROWS = 32768
COLS = 2048
BLK = 1024
NBLK = ROWS // BLK
N_TOTAL = ROWS * COLS

NW = 32
SC_ROWS = 8192
N_SC = SC_ROWS * COLS
PER_W = N_SC // NW
S = 16384
NCHUNK = PER_W // S
L = 16
TC_ROWS = ROWS - SC_ROWS


def _stats_body(x_ref, acc_ref):
    j = pl.program_id(0)

    @pl.when(j == 0)
    def _init():
        acc_ref[...] = jnp.zeros_like(acc_ref)

    blk = x_ref[...]
    g = blk.reshape(BLK // 8, 8, COLS)
    acc_ref[0] += jnp.sum(g, axis=0)
    acc_ref[1] += jnp.sum(g * g, axis=0)


def _tc_map_body(tab_ref, x_ref, o_ref):
    x = x_ref[...]
    a = jnp.full_like(x, tab_ref[1, 0])
    b = jnp.full_like(x, tab_ref[2, 0])
    for k in range(1, NBINS):
        m = x >= tab_ref[0, k]
        a = jnp.where(m, tab_ref[1, k], a)
        b = jnp.where(m, tab_ref[2, k], b)
    o_ref[...] = a * x + b


def _sc_map_body(x_hbm, offv_hbm, invv_hbm, ta_hbm, tb_hbm, out_hbm,
                 xv0, xv1, yv0, yv1, tav, tbv, offv, invv,
                 si0, si1, so0, so1):
    c = lax.axis_index("c")
    s = lax.axis_index("s")
    wid = s * 2 + c
    pltpu.sync_copy(ta_hbm, tav)
    pltpu.sync_copy(tb_hbm, tbv)
    pltpu.sync_copy(offv_hbm, offv)
    pltpu.sync_copy(invv_hbm, invv)
    off = offv[...]
    inv = invv[...]
    base = wid * PER_W

    def compute(xv, yv):
        @plsc.parallel_loop(0, S // L, 1, unroll=8)
        def _grp(i):
            xg = xv[pl.ds(i * L, L)]
            u = (xg - off) * inv
            idx = jnp.clip(u.astype(jnp.int32), 0, NBINS - 1)
            a = plsc.load_gather(tav, [idx])
            b = plsc.load_gather(tbv, [idx])
            yv[pl.ds(i * L, L)] = a * xg + b

    pltpu.async_copy(x_hbm.at[pl.ds(base, S)], xv0, si0)
    pltpu.async_copy(x_hbm.at[pl.ds(base + S, S)], xv1, si1)

    def stage(g, cidx, xv, yv, si, so):
        pltpu.make_async_copy(x_hbm.at[pl.ds(base, S)], xv, si).wait()

        @pl.when(g > 0)
        def _drain_out():
            pltpu.make_async_copy(yv, out_hbm.at[pl.ds(base, S)], so).wait()

        compute(xv, yv)
        pltpu.async_copy(yv, out_hbm.at[pl.ds(base + cidx * S, S)], so)

        @pl.when(cidx + 2 < NCHUNK)
        def _next_in():
            pltpu.async_copy(x_hbm.at[pl.ds(base + (cidx + 2) * S, S)], xv, si)

    def pair(g, carry):
        stage(g, 2 * g, xv0, yv0, si0, so0)
        stage(g, 2 * g + 1, xv1, yv1, si1, so1)
        return carry

    lax.fori_loop(0, NCHUNK // 2, pair, 0)
    pltpu.make_async_copy(yv0, out_hbm.at[pl.ds(base, S)], so0).wait()
    pltpu.make_async_copy(yv1, out_hbm.at[pl.ds(base, S)], so1).wait()


@jax.jit
def kernel(x, alphas, betas):
    init_alpha = jnp.concatenate([
        jnp.zeros((NBINS // 2,), dtype=jnp.float32),
        jnp.ones((NBINS - NBINS // 2,), dtype=jnp.float32),
    ])
    x2 = x.reshape(ROWS, COLS)

    acc = pl.pallas_call(
        _stats_body,
        grid=(NBLK,),
        in_specs=[pl.BlockSpec((BLK, COLS), lambda j: (j, 0))],
        out_specs=pl.BlockSpec((2, 8, COLS), lambda j: (0, 0, 0)),
        out_shape=jax.ShapeDtypeStruct((2, 8, COLS), jnp.float32),
    )(x2)

    n = jnp.float32(N_TOTAL)
    ssum = jnp.sum(acc[0])
    ssq = jnp.sum(acc[1])
    mean = ssum / n
    var = (ssq - ssum * ssum / n) / (n - 1.0)
    std = jnp.sqrt(var)
    step = 6.0 * std / NBINS
    off = mean - 3.0 * std
    inv = 1.0 / step
    offv = jnp.full((L,), off, jnp.float32)
    invv = jnp.full((L,), inv, jnp.float32)
    ta8 = alphas + init_alpha
    tav = jnp.pad(ta8, (0, L - NBINS))
    tbv = jnp.pad(betas, (0, L - NBINS))
    th = off + step * jnp.arange(NBINS, dtype=jnp.float32)
    tab = jnp.stack([th, ta8, betas])

    mesh = plsc.VectorSubcoreMesh(
        core_axis_name="c", subcore_axis_name="s", num_cores=2, num_subcores=16)
    out_sc = pl.kernel(
        _sc_map_body,
        out_type=jax.ShapeDtypeStruct((N_SC,), jnp.float32),
        mesh=mesh,
        compiler_params=pltpu.CompilerParams(needs_layout_passes=False),
        scratch_types=[
            pltpu.VMEM((S,), jnp.float32),
            pltpu.VMEM((S,), jnp.float32),
            pltpu.VMEM((S,), jnp.float32),
            pltpu.VMEM((S,), jnp.float32),
            pltpu.VMEM((L,), jnp.float32),
            pltpu.VMEM((L,), jnp.float32),
            pltpu.VMEM((L,), jnp.float32),
            pltpu.VMEM((L,), jnp.float32),
            pltpu.SemaphoreType.DMA,
            pltpu.SemaphoreType.DMA,
            pltpu.SemaphoreType.DMA,
            pltpu.SemaphoreType.DMA,
        ],
    )(x.reshape(N_TOTAL), offv, invv, tav, tbv)

    out_tc = pl.pallas_call(
        _tc_map_body,
        grid=(TC_ROWS // BLK,),
        in_specs=[
            pl.BlockSpec(memory_space=pltpu.SMEM),
            pl.BlockSpec((BLK, COLS), lambda j: (j + SC_ROWS // BLK, 0)),
        ],
        out_specs=pl.BlockSpec((BLK, COLS), lambda j: (j, 0)),
        out_shape=jax.ShapeDtypeStruct((TC_ROWS, COLS), jnp.float32),
    )(tab, x2)

    out = jnp.concatenate([out_sc.reshape(SC_ROWS, COLS), out_tc], axis=0)
    return out.reshape(x.shape)


# speedup vs baseline: 4.1181x; 1.4192x over previous
import functools

import jax
import jax.numpy as jnp
from jax import lax
from jax.experimental import pallas as pl
from jax.experimental.pallas import tpu as pltpu
from jax.experimental.pallas import tpu_sc as plsc

NBINS = 8
ROWS = 32768
COLS = 2048
BLK = 1024
N_TOTAL = ROWS * COLS

NW = 32
L = 16
S = 16384
SC_STAT_ROWS = 8192
N_SC = SC_STAT_ROWS * COLS
PER_W = N_SC // NW
NCHUNK = PER_W // S
TC_STAT_ROWS = ROWS - SC_STAT_ROWS
UNROLL = 8


def _tc_stats_body(x_ref, acc_ref):
    j = pl.program_id(0)

    @pl.when(j == 0)
    def _init():
        acc_ref[...] = jnp.zeros_like(acc_ref)

    blk = x_ref[...]
    g = blk.reshape(BLK // 8, 8, COLS)
    acc_ref[0] += jnp.sum(g, axis=0)
    acc_ref[1] += jnp.sum(g * g, axis=0)


def _sc_stats_body(x_hbm, outs_hbm, outq_hbm, xv0, xv1, sv, qv, si0, si1):
    c = lax.axis_index("c")
    s = lax.axis_index("s")
    wid = s * 2 + c
    base = wid * PER_W
    zero = jnp.zeros((L,), jnp.float32)
    acc0 = tuple(zero for _ in range(2 * UNROLL))

    pltpu.async_copy(x_hbm.at[pl.ds(base, S)], xv0, si0)
    pltpu.async_copy(x_hbm.at[pl.ds(base + S, S)], xv1, si1)

    def accum_chunk(xv, acc):
        def grp(i, a):
            parts = []
            for u in range(UNROLL):
                xg = xv[pl.ds((i * UNROLL + u) * L, L)]
                parts.append((a[2 * u] + xg, a[2 * u + 1] + xg * xg))
            return tuple(v for p in parts for v in p)

        return lax.fori_loop(0, S // (L * UNROLL), grp, acc)

    def stage(cidx, acc, xv, si):
        pltpu.make_async_copy(x_hbm.at[pl.ds(base, S)], xv, si).wait()
        acc = accum_chunk(xv, acc)

        @pl.when(cidx + 2 < NCHUNK)
        def _next_in():
            pltpu.async_copy(x_hbm.at[pl.ds(base + (cidx + 2) * S, S)], xv, si)

        return acc

    def pair(g, acc):
        acc = stage(2 * g, acc, xv0, si0)
        acc = stage(2 * g + 1, acc, xv1, si1)
        return acc

    acc = lax.fori_loop(0, NCHUNK // 2, pair, acc0)
    ssum = acc[0]
    ssq = acc[1]
    for u in range(1, UNROLL):
        ssum = ssum + acc[2 * u]
        ssq = ssq + acc[2 * u + 1]
    sv[...] = ssum
    qv[...] = ssq
    pltpu.sync_copy(sv, outs_hbm.at[wid])
    pltpu.sync_copy(qv, outq_hbm.at[wid])


def _tc_map_body(tab_ref, x_ref, o_ref):
    x = x_ref[...]
    a = jnp.full_like(x, tab_ref[1, 0])
    b = jnp.full_like(x, tab_ref[2, 0])
    for k in range(1, NBINS):
        m = x >= tab_ref[0, k]
        a = jnp.where(m, tab_ref[1, k], a)
        b = jnp.where(m, tab_ref[2, k], b)
    o_ref[...] = a * x + b


@jax.jit
def kernel(x, alphas, betas):
    init_alpha = jnp.concatenate([
        jnp.zeros((NBINS // 2,), dtype=jnp.float32),
        jnp.ones((NBINS - NBINS // 2,), dtype=jnp.float32),
    ])
    x2 = x.reshape(ROWS, COLS)

    mesh = plsc.VectorSubcoreMesh(
        core_axis_name="c", subcore_axis_name="s", num_cores=2, num_subcores=16)
    sc_s, sc_q = pl.kernel(
        _sc_stats_body,
        out_type=(jax.ShapeDtypeStruct((NW, L), jnp.float32),
                  jax.ShapeDtypeStruct((NW, L), jnp.float32)),
        mesh=mesh,
        compiler_params=pltpu.CompilerParams(needs_layout_passes=False),
        scratch_types=[
            pltpu.VMEM((S,), jnp.float32),
            pltpu.VMEM((S,), jnp.float32),
            pltpu.VMEM((L,), jnp.float32),
            pltpu.VMEM((L,), jnp.float32),
            pltpu.SemaphoreType.DMA,
            pltpu.SemaphoreType.DMA,
        ],
    )(x.reshape(N_TOTAL))

    acc = pl.pallas_call(
        _tc_stats_body,
        grid=(TC_STAT_ROWS // BLK,),
        in_specs=[
            pl.BlockSpec((BLK, COLS), lambda j: (j + SC_STAT_ROWS // BLK, 0))],
        out_specs=pl.BlockSpec((2, 8, COLS), lambda j: (0, 0, 0)),
        out_shape=jax.ShapeDtypeStruct((2, 8, COLS), jnp.float32),
    )(x2)

    n = jnp.float32(N_TOTAL)
    ssum = jnp.sum(acc[0]) + jnp.sum(sc_s)
    ssq = jnp.sum(acc[1]) + jnp.sum(sc_q)
    mean = ssum / n
    var = (ssq - ssum * ssum / n) / (n - 1.0)
    std = jnp.sqrt(var)
    step = 6.0 * std / NBINS
    off = mean - 3.0 * std
    th = off + step * jnp.arange(NBINS, dtype=jnp.float32)
    tab = jnp.stack([th, alphas + init_alpha, betas])

    out = pl.pallas_call(
        _tc_map_body,
        grid=(ROWS // BLK,),
        in_specs=[
            pl.BlockSpec(memory_space=pltpu.SMEM),
            pl.BlockSpec((BLK, COLS), lambda j: (j, 0)),
        ],
        out_specs=pl.BlockSpec((BLK, COLS), lambda j: (j, 0)),
        out_shape=jax.ShapeDtypeStruct((ROWS, COLS), jnp.float32),
    )(tab, x2)
    return out.reshape(x.shape)


# speedup vs baseline: 6.5510x; 1.5908x over previous
import functools

import jax
import jax.numpy as jnp
from jax import lax
from jax.experimental import pallas as pl
from jax.experimental.pallas import tpu as pltpu
from jax.experimental.pallas import tpu_sc as plsc

NBINS = 8
ROWS = 32768
COLS = 2048
BLK = 1024
N_TOTAL = ROWS * COLS

NW = 32
L = 16
SC_STAT_ROWS = 8192
RCHUNK = 8
NCHUNK = SC_STAT_ROWS // NW // RCHUNK
TC_STAT_ROWS = ROWS - SC_STAT_ROWS
UNROLL = 8


def _tc_stats_body(x_ref, acc_ref):
    j = pl.program_id(0)

    @pl.when(j == 0)
    def _init():
        acc_ref[...] = jnp.zeros_like(acc_ref)

    blk = x_ref[...]
    g = blk.reshape(BLK // 8, 8, COLS)
    acc_ref[0] += jnp.sum(g, axis=0)
    acc_ref[1] += jnp.sum(g * g, axis=0)


def _sc_stats_body(x_hbm, outs_hbm, outq_hbm, xv0, xv1, sv, qv, si0, si1):
    c = lax.axis_index("c")
    s = lax.axis_index("s")
    wid = s * 2 + c
    base = wid * (SC_STAT_ROWS // NW)
    zero = jnp.zeros((L,), jnp.float32)
    acc0 = tuple(zero for _ in range(2 * UNROLL))

    pltpu.async_copy(x_hbm.at[pl.ds(base, RCHUNK)], xv0, si0)
    pltpu.async_copy(x_hbm.at[pl.ds(base + RCHUNK, RCHUNK)], xv1, si1)

    def accum_chunk(xv, acc):
        def grp(i, a):
            parts = []
            for u in range(UNROLL):
                xg = xv[u, pl.ds(i * L, L)]
                parts.append((a[2 * u] + xg, a[2 * u + 1] + xg * xg))
            return tuple(v for p in parts for v in p)

        return lax.fori_loop(0, COLS // L, grp, acc)

    def stage(cidx, acc, xv, si):
        pltpu.make_async_copy(x_hbm.at[pl.ds(base, RCHUNK)], xv, si).wait()
        acc = accum_chunk(xv, acc)

        @pl.when(cidx + 2 < NCHUNK)
        def _next_in():
            pltpu.async_copy(
                x_hbm.at[pl.ds(base + (cidx + 2) * RCHUNK, RCHUNK)], xv, si)

        return acc

    def pair(g, acc):
        acc = stage(2 * g, acc, xv0, si0)
        acc = stage(2 * g + 1, acc, xv1, si1)
        return acc

    acc = lax.fori_loop(0, NCHUNK // 2, pair, acc0)
    ssum = acc[0]
    ssq = acc[1]
    for u in range(1, UNROLL):
        ssum = ssum + acc[2 * u]
        ssq = ssq + acc[2 * u + 1]
    sv[...] = ssum
    qv[...] = ssq
    pltpu.sync_copy(sv, outs_hbm.at[wid])
    pltpu.sync_copy(qv, outq_hbm.at[wid])


def _tc_map_body(tab_ref, x_ref, o_ref):
    x = x_ref[...]
    a = jnp.full_like(x, tab_ref[1, 0])
    b = jnp.full_like(x, tab_ref[2, 0])
    for k in range(1, NBINS):
        m = x >= tab_ref[0, k]
        a = jnp.where(m, tab_ref[1, k], a)
        b = jnp.where(m, tab_ref[2, k], b)
    o_ref[...] = a * x + b


@jax.jit
def kernel(x, alphas, betas):
    init_alpha = jnp.concatenate([
        jnp.zeros((NBINS // 2,), dtype=jnp.float32),
        jnp.ones((NBINS - NBINS // 2,), dtype=jnp.float32),
    ])
    x2 = x.reshape(ROWS, COLS)

    mesh = plsc.VectorSubcoreMesh(
        core_axis_name="c", subcore_axis_name="s", num_cores=2, num_subcores=16)
    sc_s, sc_q = pl.kernel(
        _sc_stats_body,
        out_type=(jax.ShapeDtypeStruct((NW, L), jnp.float32),
                  jax.ShapeDtypeStruct((NW, L), jnp.float32)),
        mesh=mesh,
        compiler_params=pltpu.CompilerParams(needs_layout_passes=False),
        scratch_types=[
            pltpu.VMEM((RCHUNK, COLS), jnp.float32),
            pltpu.VMEM((RCHUNK, COLS), jnp.float32),
            pltpu.VMEM((L,), jnp.float32),
            pltpu.VMEM((L,), jnp.float32),
            pltpu.SemaphoreType.DMA,
            pltpu.SemaphoreType.DMA,
        ],
    )(x2)

    acc = pl.pallas_call(
        _tc_stats_body,
        grid=(TC_STAT_ROWS // BLK,),
        in_specs=[
            pl.BlockSpec((BLK, COLS), lambda j: (j + SC_STAT_ROWS // BLK, 0))],
        out_specs=pl.BlockSpec((2, 8, COLS), lambda j: (0, 0, 0)),
        out_shape=jax.ShapeDtypeStruct((2, 8, COLS), jnp.float32),
    )(x2)

    n = jnp.float32(N_TOTAL)
    ssum = jnp.sum(acc[0]) + jnp.sum(sc_s)
    ssq = jnp.sum(acc[1]) + jnp.sum(sc_q)
    mean = ssum / n
    var = (ssq - ssum * ssum / n) / (n - 1.0)
    std = jnp.sqrt(var)
    step = 6.0 * std / NBINS
    off = mean - 3.0 * std
    th = off + step * jnp.arange(NBINS, dtype=jnp.float32)
    tab = jnp.stack([th, alphas + init_alpha, betas])

    out = pl.pallas_call(
        _tc_map_body,
        grid=(ROWS // BLK,),
        in_specs=[
            pl.BlockSpec(memory_space=pltpu.SMEM),
            pl.BlockSpec((BLK, COLS), lambda j: (j, 0)),
        ],
        out_specs=pl.BlockSpec((BLK, COLS), lambda j: (j, 0)),
        out_shape=jax.ShapeDtypeStruct((ROWS, COLS), jnp.float32),
    )(tab, x2)
    return out.reshape(x.shape)


# speedup vs baseline: 7.0030x; 1.0690x over previous
import functools

import jax
import jax.numpy as jnp
from jax import lax
from jax.experimental import pallas as pl
from jax.experimental.pallas import tpu as pltpu
from jax.experimental.pallas import tpu_sc as plsc

NBINS = 8
ROWS = 32768
COLS = 2048
BLK = 1024
N_TOTAL = ROWS * COLS

NW = 32
L = 16
SC_STAT_ROWS = 0
RCHUNK = 8
NCHUNK = SC_STAT_ROWS // NW // RCHUNK
TC_STAT_ROWS = ROWS - SC_STAT_ROWS
UNROLL = 8


def _tc_stats_body(x_ref, acc_ref):
    j = pl.program_id(0)

    @pl.when(j == 0)
    def _init():
        acc_ref[...] = jnp.zeros_like(acc_ref)

    blk = x_ref[...]
    g = blk.reshape(BLK // 8, 8, COLS)
    acc_ref[0] += jnp.sum(g, axis=0)
    acc_ref[1] += jnp.sum(g * g, axis=0)


def _sc_stats_body(x_hbm, outs_hbm, outq_hbm, xv0, xv1, sv, qv, si0, si1):
    c = lax.axis_index("c")
    s = lax.axis_index("s")
    wid = s * 2 + c
    base = wid * (SC_STAT_ROWS // NW)
    zero = jnp.zeros((L,), jnp.float32)
    acc0 = tuple(zero for _ in range(2 * UNROLL))

    pltpu.async_copy(x_hbm.at[pl.ds(base, RCHUNK)], xv0, si0)
    pltpu.async_copy(x_hbm.at[pl.ds(base + RCHUNK, RCHUNK)], xv1, si1)

    def accum_chunk(xv, acc):
        def grp(i, a):
            parts = []
            for u in range(UNROLL):
                xg = xv[u, pl.ds(i * L, L)]
                parts.append((a[2 * u] + xg, a[2 * u + 1] + xg * xg))
            return tuple(v for p in parts for v in p)

        return lax.fori_loop(0, COLS // L, grp, acc)

    def stage(cidx, acc, xv, si):
        pltpu.make_async_copy(x_hbm.at[pl.ds(base, RCHUNK)], xv, si).wait()
        acc = accum_chunk(xv, acc)

        @pl.when(cidx + 2 < NCHUNK)
        def _next_in():
            pltpu.async_copy(
                x_hbm.at[pl.ds(base + (cidx + 2) * RCHUNK, RCHUNK)], xv, si)

        return acc

    def pair(g, acc):
        acc = stage(2 * g, acc, xv0, si0)
        acc = stage(2 * g + 1, acc, xv1, si1)
        return acc

    acc = lax.fori_loop(0, NCHUNK // 2, pair, acc0)
    ssum = acc[0]
    ssq = acc[1]
    for u in range(1, UNROLL):
        ssum = ssum + acc[2 * u]
        ssq = ssq + acc[2 * u + 1]
    sv[...] = ssum
    qv[...] = ssq
    pltpu.sync_copy(sv, outs_hbm.at[wid])
    pltpu.sync_copy(qv, outq_hbm.at[wid])


def _tc_map_body(tab_ref, x_ref, o_ref):
    x = x_ref[...]
    a = jnp.full_like(x, tab_ref[1, 0])
    b = jnp.full_like(x, tab_ref[2, 0])
    for k in range(1, NBINS):
        m = x >= tab_ref[0, k]
        a = jnp.where(m, tab_ref[1, k], a)
        b = jnp.where(m, tab_ref[2, k], b)
    o_ref[...] = a * x + b


@jax.jit
def kernel(x, alphas, betas):
    init_alpha = jnp.concatenate([
        jnp.zeros((NBINS // 2,), dtype=jnp.float32),
        jnp.ones((NBINS - NBINS // 2,), dtype=jnp.float32),
    ])
    x2 = x.reshape(ROWS, COLS)


    acc = pl.pallas_call(
        _tc_stats_body,
        grid=(TC_STAT_ROWS // BLK,),
        in_specs=[
            pl.BlockSpec((BLK, COLS), lambda j: (j + SC_STAT_ROWS // BLK, 0))],
        out_specs=pl.BlockSpec((2, 8, COLS), lambda j: (0, 0, 0)),
        out_shape=jax.ShapeDtypeStruct((2, 8, COLS), jnp.float32),
    )(x2)

    n = jnp.float32(N_TOTAL)
    ssum = jnp.sum(acc[0])
    ssq = jnp.sum(acc[1])
    mean = ssum / n
    var = (ssq - ssum * ssum / n) / (n - 1.0)
    std = jnp.sqrt(var)
    step = 6.0 * std / NBINS
    off = mean - 3.0 * std
    th = off + step * jnp.arange(NBINS, dtype=jnp.float32)
    tab = jnp.stack([th, alphas + init_alpha, betas])

    out = pl.pallas_call(
        _tc_map_body,
        grid=(ROWS // BLK,),
        in_specs=[
            pl.BlockSpec(memory_space=pltpu.SMEM),
            pl.BlockSpec((BLK, COLS), lambda j: (j, 0)),
        ],
        out_specs=pl.BlockSpec((BLK, COLS), lambda j: (j, 0)),
        out_shape=jax.ShapeDtypeStruct((ROWS, COLS), jnp.float32),
    )(tab, x2)
    return out.reshape(x.shape)


# speedup vs baseline: 7.1383x; 1.0193x over previous
import functools

import jax
import jax.numpy as jnp
from jax.experimental import pallas as pl
from jax.experimental.pallas import tpu as pltpu

NBINS = 8
ROWS = 32768
COLS = 2048
BLK = 1024
NBLK = ROWS // BLK
N_TOTAL = ROWS * COLS


def _body(ab_ref, x_ref, o_ref, acc_ref, stat_ref):
    phase = pl.program_id(0)
    j = pl.program_id(1)

    @pl.when(jnp.logical_and(phase == 0, j == 0))
    def _init():
        acc_ref[...] = jnp.zeros_like(acc_ref)

    @pl.when(phase == 0)
    def _stats():
        blk = x_ref[...]
        g = blk.reshape(BLK // 8, 8, COLS)
        acc_ref[0] += jnp.sum(g, axis=0)
        acc_ref[1] += jnp.sum(g * g, axis=0)

    @pl.when(jnp.logical_and(phase == 1, j == 0))
    def _finalize():
        n = jnp.float32(N_TOTAL)
        s = jnp.sum(acc_ref[0])
        s2 = jnp.sum(acc_ref[1])
        mean = s / n
        var = (s2 - s * s / n) / (n - 1.0)
        std = jnp.sqrt(var)
        step = 6.0 * std / NBINS
        off = mean - 3.0 * std
        for k in range(1, NBINS):
            stat_ref[k] = off + jnp.float32(k) * step

    @pl.when(phase == 1)
    def _map():
        x = x_ref[...]
        a = jnp.full_like(x, ab_ref[0, 0])
        b = jnp.full_like(x, ab_ref[1, 0])
        for k in range(1, NBINS):
            m = x >= stat_ref[k]
            a = jnp.where(m, ab_ref[0, k], a)
            b = jnp.where(m, ab_ref[1, k], b)
        o_ref[...] = a * x + b


@jax.jit
def kernel(x, alphas, betas):
    init_alpha = jnp.concatenate([
        jnp.zeros((NBINS // 2,), dtype=jnp.float32),
        jnp.ones((NBINS - NBINS // 2,), dtype=jnp.float32),
    ])
    ab = jnp.stack([alphas + init_alpha, betas])
    x2 = x.reshape(ROWS, COLS)
    out = pl.pallas_call(
        _body,
        grid=(2, NBLK),
        in_specs=[
            pl.BlockSpec(memory_space=pltpu.SMEM),
            pl.BlockSpec((BLK, COLS), lambda p, j: (j, 0)),
        ],
        out_specs=pl.BlockSpec((BLK, COLS), lambda p, j: (p * j, 0)),
        out_shape=jax.ShapeDtypeStruct((ROWS, COLS), jnp.float32),
        scratch_shapes=[
            pltpu.VMEM((2, 8, COLS), jnp.float32),
            pltpu.SMEM((NBINS,), jnp.float32),
        ],
    )(ab, x2)
    return out.reshape(x.shape)
